# baseline (speedup 1.0000x reference)
import jax
import jax.numpy as jnp
from jax import lax
from jax.experimental import pallas as pl
from jax.experimental.pallas import tpu as pltpu
from jax.experimental.pallas import tpu_sc as plsc

D = 128
LANES = 16
VPR = D // LANES
NCORES = 2
NSUB = 16
NW = NCORES * NSUB
CHUNK = 128
MAGIC = 8388608.0
INV7 = 1.0 / 7.0
TINY = 1e-30


def _compute_chunk(in_v, out_v):
    @plsc.parallel_loop(0, CHUNK, unroll=2)
    def _(r):
        vs = [in_v[r, pl.ds(j * LANES, LANES)] for j in range(VPR)]
        mn = jnp.minimum(jnp.minimum(jnp.minimum(vs[0], vs[1]),
                                     jnp.minimum(vs[2], vs[3])),
                         jnp.minimum(jnp.minimum(vs[4], vs[5]),
                                     jnp.minimum(vs[6], vs[7])))
        mx = jnp.maximum(jnp.maximum(jnp.maximum(vs[0], vs[1]),
                                     jnp.maximum(vs[2], vs[3])),
                         jnp.maximum(jnp.maximum(vs[4], vs[5]),
                                     jnp.maximum(vs[6], vs[7])))
        mn = jnp.full((LANES,), jnp.min(mn), jnp.float32)
        mx = jnp.full((LANES,), jnp.max(mx), jnp.float32)
        safe = jnp.maximum(mx - mn, TINY)
        inv = 7.0 / safe
        scale = safe * INV7
        for j in range(VPR):
            c = (vs[j] - mn) * inv
            rnd = (c + MAGIC) - MAGIC
            out_v[r, pl.ds(j * LANES, LANES)] = rnd * scale + mn


def _sc_body(x_hbm, o_hbm, in0, in1, out0, out1,
             si0, si1, so0, so1):
    rows_per_w = x_hbm.shape[0] // NW
    wid = lax.axis_index("s") * NCORES + lax.axis_index("c")
    base = wid * rows_per_w
    nchunks = rows_per_w // CHUNK
    npairs = nchunks // 2

    def start_in(g, buf, sem):
        pltpu.async_copy(x_hbm.at[pl.ds(base + g * CHUNK, CHUNK)], buf, sem)

    def wait_in(buf, sem):
        pltpu.make_async_copy(x_hbm.at[pl.ds(base, CHUNK)], buf, sem).wait()

    def start_out(g, buf, sem):
        pltpu.async_copy(buf, o_hbm.at[pl.ds(base + g * CHUNK, CHUNK)], sem)

    def wait_out(buf, sem):
        pltpu.make_async_copy(buf, o_hbm.at[pl.ds(base, CHUNK)], sem).wait()

    start_in(0, in0, si0)
    start_in(1, in1, si1)

    wait_in(in0, si0)
    _compute_chunk(in0, out0)
    start_out(0, out0, so0)
    start_in(2, in0, si0)

    wait_in(in1, si1)
    _compute_chunk(in1, out1)
    start_out(1, out1, so1)
    start_in(3, in1, si1)

    def pair(p, carry):
        g = 2 * p

        def phase(g, in_b, out_b, si, so):
            wait_in(in_b, si)
            wait_out(out_b, so)
            _compute_chunk(in_b, out_b)
            start_out(g, out_b, so)

            @pl.when(g + 2 < nchunks)
            def _():
                start_in(g + 2, in_b, si)

        phase(g, in0, out0, si0, so0)
        phase(g + 1, in1, out1, si1, so1)
        return carry

    lax.fori_loop(1, npairs, pair, 0)
    wait_out(out0, so0)
    wait_out(out1, so1)


def _quantize_recon(x):
    n = x.shape[0]
    mesh = plsc.VectorSubcoreMesh(
        core_axis_name="c", subcore_axis_name="s",
        num_cores=NCORES, num_subcores=NSUB)
    return pl.kernel(
        _sc_body,
        out_type=jax.ShapeDtypeStruct((n, D), jnp.float32),
        mesh=mesh,
        scratch_types=[
            pltpu.VMEM((CHUNK, D), jnp.float32),
            pltpu.VMEM((CHUNK, D), jnp.float32),
            pltpu.VMEM((CHUNK, D), jnp.float32),
            pltpu.VMEM((CHUNK, D), jnp.float32),
            pltpu.SemaphoreType.DMA,
            pltpu.SemaphoreType.DMA,
            pltpu.SemaphoreType.DMA,
            pltpu.SemaphoreType.DMA,
        ],
        compiler_params=pltpu.CompilerParams(needs_layout_passes=False),
    )(x)


def kernel(kv_states):
    batch, num_heads, seq_len, head_dim = kv_states.shape
    x = kv_states.astype(jnp.float32).reshape(-1, head_dim)
    recon = _quantize_recon(x)
    return recon.reshape(batch, num_heads, seq_len, head_dim)


# speedup vs baseline: 1.0320x; 1.0320x over previous
import jax
import jax.numpy as jnp
from jax import lax
from jax.experimental import pallas as pl
from jax.experimental.pallas import tpu as pltpu
from jax.experimental.pallas import tpu_sc as plsc

D = 128
LANES = 16
VPR = D // LANES
NCORES = 2
NSUB = 16
NW = NCORES * NSUB
CHUNK = 128
MAGIC = 8388608.0
INV7 = 1.0 / 7.0
TINY = 1e-30


def _compute_chunk(in_v, out_v):
    @plsc.parallel_loop(0, CHUNK, unroll=2)
    def _(r):
        vs = [in_v[r, pl.ds(j * LANES, LANES)] for j in range(VPR)]
        mn = jnp.minimum(jnp.minimum(jnp.minimum(vs[0], vs[1]),
                                     jnp.minimum(vs[2], vs[3])),
                         jnp.minimum(jnp.minimum(vs[4], vs[5]),
                                     jnp.minimum(vs[6], vs[7])))
        mx = jnp.maximum(jnp.maximum(jnp.maximum(vs[0], vs[1]),
                                     jnp.maximum(vs[2], vs[3])),
                         jnp.maximum(jnp.maximum(vs[4], vs[5]),
                                     jnp.maximum(vs[6], vs[7])))
        mn = jnp.full((LANES,), jnp.min(mn), jnp.float32)
        mx = jnp.full((LANES,), jnp.max(mx), jnp.float32)
        safe = jnp.maximum(mx - mn, TINY)
        inv = 7.0 / safe
        scale = safe * INV7
        for j in range(VPR):
            c = (vs[j] - mn) * inv
            rnd = (c + MAGIC) - MAGIC
            sl = pl.ds(j * LANES, LANES)
            out_v[r, sl] = rnd * scale
            plsc.addupdate(out_v.at[r, sl], mn)


def _sc_body(x_hbm, o_hbm, in0, in1, out0, out1,
             si0, si1, so0, so1):
    rows_per_w = x_hbm.shape[0] // NW
    wid = lax.axis_index("s") * NCORES + lax.axis_index("c")
    base = wid * rows_per_w
    nchunks = rows_per_w // CHUNK
    npairs = nchunks // 2

    def start_in(g, buf, sem):
        pltpu.async_copy(x_hbm.at[pl.ds(base + g * CHUNK, CHUNK)], buf, sem)

    def wait_in(buf, sem):
        pltpu.make_async_copy(x_hbm.at[pl.ds(base, CHUNK)], buf, sem).wait()

    def start_out(g, buf, sem):
        pltpu.async_copy(buf, o_hbm.at[pl.ds(base + g * CHUNK, CHUNK)], sem)

    def wait_out(buf, sem):
        pltpu.make_async_copy(buf, o_hbm.at[pl.ds(base, CHUNK)], sem).wait()

    start_in(0, in0, si0)
    start_in(1, in1, si1)

    wait_in(in0, si0)
    _compute_chunk(in0, out0)
    start_out(0, out0, so0)
    start_in(2, in0, si0)

    wait_in(in1, si1)
    _compute_chunk(in1, out1)
    start_out(1, out1, so1)
    start_in(3, in1, si1)

    def pair(p, carry):
        g = 2 * p

        def phase(g, in_b, out_b, si, so):
            wait_in(in_b, si)
            wait_out(out_b, so)
            _compute_chunk(in_b, out_b)
            start_out(g, out_b, so)

            @pl.when(g + 2 < nchunks)
            def _():
                start_in(g + 2, in_b, si)

        phase(g, in0, out0, si0, so0)
        phase(g + 1, in1, out1, si1, so1)
        return carry

    lax.fori_loop(1, npairs, pair, 0)
    wait_out(out0, so0)
    wait_out(out1, so1)


def _quantize_recon(x):
    n = x.shape[0]
    mesh = plsc.VectorSubcoreMesh(
        core_axis_name="c", subcore_axis_name="s",
        num_cores=NCORES, num_subcores=NSUB)
    return pl.kernel(
        _sc_body,
        out_type=jax.ShapeDtypeStruct((n, D), jnp.float32),
        mesh=mesh,
        scratch_types=[
            pltpu.VMEM((CHUNK, D), jnp.float32),
            pltpu.VMEM((CHUNK, D), jnp.float32),
            pltpu.VMEM((CHUNK, D), jnp.float32),
            pltpu.VMEM((CHUNK, D), jnp.float32),
            pltpu.SemaphoreType.DMA,
            pltpu.SemaphoreType.DMA,
            pltpu.SemaphoreType.DMA,
            pltpu.SemaphoreType.DMA,
        ],
        compiler_params=pltpu.CompilerParams(
            needs_layout_passes=False,
            disable_bounds_checks=True,
            disable_semaphore_checks=True,
        ),
    )(x)


def kernel(kv_states):
    batch, num_heads, seq_len, head_dim = kv_states.shape
    x = kv_states.astype(jnp.float32).reshape(-1, head_dim)
    recon = _quantize_recon(x)
    return recon.reshape(batch, num_heads, seq_len, head_dim)


# speedup vs baseline: 1.0346x; 1.0026x over previous
import jax
import jax.numpy as jnp
from jax import lax
from jax.experimental import pallas as pl
from jax.experimental.pallas import tpu as pltpu
from jax.experimental.pallas import tpu_sc as plsc

D = 128
LANES = 16
VPR = D // LANES
NCORES = 2
NSUB = 16
NW = NCORES * NSUB
CHUNK = 128
MAGIC = 8388608.0
INV7 = 1.0 / 7.0
TINY = 1e-30


def _compute_chunk(in_v, out_v):
    @plsc.parallel_loop(0, CHUNK, unroll=2)
    def _(r):
        vs = [in_v[r, pl.ds(j * LANES, LANES)] for j in range(VPR)]
        mn = jnp.minimum(jnp.minimum(jnp.minimum(vs[0], vs[1]),
                                     jnp.minimum(vs[2], vs[3])),
                         jnp.minimum(jnp.minimum(vs[4], vs[5]),
                                     jnp.minimum(vs[6], vs[7])))
        mx = jnp.maximum(jnp.maximum(jnp.maximum(vs[0], vs[1]),
                                     jnp.maximum(vs[2], vs[3])),
                         jnp.maximum(jnp.maximum(vs[4], vs[5]),
                                     jnp.maximum(vs[6], vs[7])))
        mn = jnp.full((LANES,), jnp.min(mn), jnp.float32)
        mx = jnp.full((LANES,), jnp.max(mx), jnp.float32)
        safe = jnp.maximum(mx - mn, TINY)
        inv = 7.0 / safe
        scale = safe * INV7
        for j in range(VPR):
            c = (vs[j] - mn) * inv
            rnd = (c + MAGIC) - MAGIC
            sl = pl.ds(j * LANES, LANES)
            out_v[r, sl] = rnd * scale
            plsc.addupdate(out_v.at[r, sl], mn)


def _sc_body(x_hbm, o_hbm, in0, in1, out0, out1,
             si0, si1, so0, so1):
    rows_per_w = x_hbm.shape[0] // NW
    wid = lax.axis_index("s") * NCORES + lax.axis_index("c")
    base = wid * rows_per_w
    nchunks = rows_per_w // CHUNK
    npairs = nchunks // 2

    def start_in(g, buf, sem):
        pltpu.async_copy(x_hbm.at[pl.ds(base + g * CHUNK, CHUNK)], buf, sem)

    def wait_in(buf, sem):
        pltpu.make_async_copy(x_hbm.at[pl.ds(base, CHUNK)], buf, sem).wait()

    def start_out(g, buf, sem):
        pltpu.async_copy(buf, o_hbm.at[pl.ds(base + g * CHUNK, CHUNK)], sem)

    def wait_out(buf, sem):
        pltpu.make_async_copy(buf, o_hbm.at[pl.ds(base, CHUNK)], sem).wait()

    start_in(0, in0, si0)
    start_in(1, in1, si1)

    wait_in(in0, si0)
    _compute_chunk(in0, out0)
    start_out(0, out0, so0)
    start_in(2, in0, si0)

    wait_in(in1, si1)
    _compute_chunk(in1, out1)
    start_out(1, out1, so1)
    start_in(3, in1, si1)

    def pair(p, carry):
        g = 2 * p

        def phase(g, in_b, out_b, si, so):
            wait_in(in_b, si)
            wait_out(out_b, so)
            _compute_chunk(in_b, out_b)
            start_out(g, out_b, so)

            @pl.when(g + 2 < nchunks)
            def _():
                start_in(g + 2, in_b, si)

        phase(g, in0, out0, si0, so0)
        phase(g + 1, in1, out1, si1, so1)
        return carry

    lax.fori_loop(1, npairs, pair, 0)
    wait_out(out0, so0)
    wait_out(out1, so1)


def _quantize_recon(x):
    n = x.shape[0]
    mesh = plsc.VectorSubcoreMesh(
        core_axis_name="c", subcore_axis_name="s",
        num_cores=NCORES, num_subcores=NSUB)
    return pl.kernel(
        _sc_body,
        out_type=jax.ShapeDtypeStruct((n, D), jnp.float32),
        mesh=mesh,
        scratch_types=[
            pltpu.VMEM((CHUNK, D), jnp.float32),
            pltpu.VMEM((CHUNK, D), jnp.float32),
            pltpu.VMEM((CHUNK, D), jnp.float32),
            pltpu.VMEM((CHUNK, D), jnp.float32),
            pltpu.SemaphoreType.DMA,
            pltpu.SemaphoreType.DMA,
            pltpu.SemaphoreType.DMA,
            pltpu.SemaphoreType.DMA,
        ],
        compiler_params=pltpu.CompilerParams(needs_layout_passes=False),
    )(x)


def kernel(kv_states):
    batch, num_heads, seq_len, head_dim = kv_states.shape
    x = kv_states.astype(jnp.float32).reshape(-1, head_dim)
    recon = _quantize_recon(x)
    return recon.reshape(batch, num_heads, seq_len, head_dim)


# speedup vs baseline: 1.0451x; 1.0101x over previous
import jax
import jax.numpy as jnp
from jax import lax
from jax.experimental import pallas as pl
from jax.experimental.pallas import tpu as pltpu
from jax.experimental.pallas import tpu_sc as plsc

D = 128
LANES = 16
VPR = D // LANES
NCORES = 2
NSUB = 16
NW = NCORES * NSUB
CHUNK = 128
MAGIC = 8388608.0
INV7 = 1.0 / 7.0
TINY = 1e-30


def _compute_chunk(in_v, out_v):
    @plsc.parallel_loop(0, CHUNK, unroll=2)
    def _(r):
        vs = [in_v[r, pl.ds(j * LANES, LANES)] for j in range(VPR)]
        mn = jnp.minimum(jnp.minimum(jnp.minimum(vs[0], vs[1]),
                                     jnp.minimum(vs[2], vs[3])),
                         jnp.minimum(jnp.minimum(vs[4], vs[5]),
                                     jnp.minimum(vs[6], vs[7])))
        mx = jnp.maximum(jnp.maximum(jnp.maximum(vs[0], vs[1]),
                                     jnp.maximum(vs[2], vs[3])),
                         jnp.maximum(jnp.maximum(vs[4], vs[5]),
                                     jnp.maximum(vs[6], vs[7])))
        mn = jnp.full((LANES,), jnp.min(mn), jnp.float32)
        mx = jnp.full((LANES,), jnp.max(mx), jnp.float32)
        safe = jnp.maximum(mx - mn, TINY)
        inv = 7.0 / safe
        scale = safe * INV7
        for j in range(VPR):
            c = (vs[j] - mn) * inv
            rnd = (c + MAGIC) - MAGIC
            sl = pl.ds(j * LANES, LANES)
            out_v[r, sl] = rnd * scale
            plsc.addupdate(out_v.at[r, sl], mn)


def _sc_body(x_hbm, o_hbm, in0, in1, out0, out1,
             si0, si1, so0, so1):
    rows_per_w = x_hbm.shape[0] // NW
    wid = lax.axis_index("s") * NCORES + lax.axis_index("c")
    base = wid * rows_per_w
    nchunks = rows_per_w // CHUNK
    npairs = nchunks // 2

    def start_in(g, buf, sem):
        pltpu.async_copy(x_hbm.at[pl.ds(base + g * CHUNK, CHUNK)], buf, sem)

    def wait_in(buf, sem):
        pltpu.make_async_copy(x_hbm.at[pl.ds(base, CHUNK)], buf, sem).wait()

    def start_out(g, buf, sem):
        pltpu.async_copy(buf, o_hbm.at[pl.ds(base + g * CHUNK, CHUNK)], sem)

    def wait_out(buf, sem):
        pltpu.make_async_copy(buf, o_hbm.at[pl.ds(base, CHUNK)], sem).wait()

    start_in(0, in0, si0)
    start_in(1, in1, si1)

    def pair(p, carry):
        g = 2 * p

        def phase(g, in_b, out_b, si, so):
            wait_in(in_b, si)

            @pl.when(g >= 2)
            def _():
                wait_out(out_b, so)

            _compute_chunk(in_b, out_b)
            start_out(g, out_b, so)

            @pl.when(g + 2 < nchunks)
            def _():
                start_in(g + 2, in_b, si)

        phase(g, in0, out0, si0, so0)
        phase(g + 1, in1, out1, si1, so1)
        return carry

    lax.fori_loop(0, npairs, pair, 0)
    wait_out(out0, so0)
    wait_out(out1, so1)


def _quantize_recon(x):
    n = x.shape[0]
    mesh = plsc.VectorSubcoreMesh(
        core_axis_name="c", subcore_axis_name="s",
        num_cores=NCORES, num_subcores=NSUB)
    return pl.kernel(
        _sc_body,
        out_type=jax.ShapeDtypeStruct((n, D), jnp.float32),
        mesh=mesh,
        scratch_types=[
            pltpu.VMEM((CHUNK, D), jnp.float32),
            pltpu.VMEM((CHUNK, D), jnp.float32),
            pltpu.VMEM((CHUNK, D), jnp.float32),
            pltpu.VMEM((CHUNK, D), jnp.float32),
            pltpu.SemaphoreType.DMA,
            pltpu.SemaphoreType.DMA,
            pltpu.SemaphoreType.DMA,
            pltpu.SemaphoreType.DMA,
        ],
        compiler_params=pltpu.CompilerParams(needs_layout_passes=False),
    )(x)


def kernel(kv_states):
    batch, num_heads, seq_len, head_dim = kv_states.shape
    x = kv_states.astype(jnp.float32).reshape(-1, head_dim)
    recon = _quantize_recon(x)
    return recon.reshape(batch, num_heads, seq_len, head_dim)


# speedup vs baseline: 1.0470x; 1.0019x over previous
import jax
import jax.numpy as jnp
from jax import lax
from jax.experimental import pallas as pl
from jax.experimental.pallas import tpu as pltpu
from jax.experimental.pallas import tpu_sc as plsc

D = 128
LANES = 16
VPR = D // LANES
NCORES = 2
NSUB = 16
NW = NCORES * NSUB
CHUNK = 128
MAGIC = 8388608.0
INV7 = 1.0 / 7.0
TINY = 1e-30


def _compute_chunk(in_v, out_v):
    @plsc.parallel_loop(0, CHUNK, unroll=2)
    def _(r):
        vs = [in_v[r, pl.ds(j * LANES, LANES)] for j in range(VPR)]
        mn = jnp.minimum(jnp.minimum(jnp.minimum(vs[0], vs[1]),
                                     jnp.minimum(vs[2], vs[3])),
                         jnp.minimum(jnp.minimum(vs[4], vs[5]),
                                     jnp.minimum(vs[6], vs[7])))
        mx = jnp.maximum(jnp.maximum(jnp.maximum(vs[0], vs[1]),
                                     jnp.maximum(vs[2], vs[3])),
                         jnp.maximum(jnp.maximum(vs[4], vs[5]),
                                     jnp.maximum(vs[6], vs[7])))
        mn = jnp.full((LANES,), jnp.min(mn), jnp.float32)
        mx = jnp.full((LANES,), jnp.max(mx), jnp.float32)
        safe = jnp.maximum(mx - mn, TINY)
        inv = 7.0 / safe
        scale = safe * INV7
        for j in range(VPR):
            c = (vs[j] - mn) * inv
            rnd = (c + MAGIC) - MAGIC
            sl = pl.ds(j * LANES, LANES)
            out_v[r, sl] = rnd * scale
            plsc.addupdate(out_v.at[r, sl], mn)


def _sc_body(x_hbm, o_hbm, in0, in1, out0, out1,
             si0, si1, so0, so1):
    rows_per_w = x_hbm.shape[0] // NW
    wid = lax.axis_index("s") * NCORES + lax.axis_index("c")
    base = wid * rows_per_w
    nchunks = rows_per_w // CHUNK
    npairs = nchunks // 2

    def start_in(g, buf, sem):
        pltpu.async_copy(x_hbm.at[pl.ds(base + g * CHUNK, CHUNK)], buf, sem)

    def wait_in(buf, sem):
        pltpu.make_async_copy(x_hbm.at[pl.ds(base, CHUNK)], buf, sem).wait()

    def start_out(g, buf, sem):
        pltpu.async_copy(buf, o_hbm.at[pl.ds(base + g * CHUNK, CHUNK)], sem)

    def wait_out(buf, sem):
        pltpu.make_async_copy(buf, o_hbm.at[pl.ds(base, CHUNK)], sem).wait()

    start_in(0, in0, si0)
    start_in(1, in1, si1)

    def pair(p, carry):
        g = 2 * p

        def phase(g, in_b, out_b, si, so):
            wait_in(in_b, si)

            @pl.when(g >= 2)
            def _():
                wait_out(out_b, so)

            _compute_chunk(in_b, out_b)
            start_out(g, out_b, so)

            @pl.when(g + 2 < nchunks)
            def _():
                start_in(g + 2, in_b, si)

        phase(g, in0, out0, si0, so0)
        phase(g + 1, in1, out1, si1, so1)
        return carry

    lax.fori_loop(0, npairs, pair, 0)
    wait_out(out0, so0)
    wait_out(out1, so1)


def _quantize_recon(x):
    n = x.shape[0]
    mesh = plsc.VectorSubcoreMesh(
        core_axis_name="c", subcore_axis_name="s",
        num_cores=NCORES, num_subcores=NSUB)
    return pl.kernel(
        _sc_body,
        out_type=jax.ShapeDtypeStruct((n, D), jnp.float32),
        mesh=mesh,
        scratch_types=[
            pltpu.VMEM((CHUNK, D), jnp.float32),
            pltpu.VMEM((CHUNK, D), jnp.float32),
            pltpu.VMEM((CHUNK, D), jnp.float32),
            pltpu.VMEM((CHUNK, D), jnp.float32),
            pltpu.SemaphoreType.DMA,
            pltpu.SemaphoreType.DMA,
            pltpu.SemaphoreType.DMA,
            pltpu.SemaphoreType.DMA,
        ],
        compiler_params=pltpu.CompilerParams(
            needs_layout_passes=False, skip_device_barrier=True),
    )(x)


def kernel(kv_states):
    batch, num_heads, seq_len, head_dim = kv_states.shape
    x = kv_states.astype(jnp.float32).reshape(-1, head_dim)
    recon = _quantize_recon(x)
    return recon.reshape(batch, num_heads, seq_len, head_dim)


# speedup vs baseline: 1.0515x; 1.0043x over previous
import jax
import jax.numpy as jnp
from jax import lax
from jax.experimental import pallas as pl
from jax.experimental.pallas import tpu as pltpu
from jax.experimental.pallas import tpu_sc as plsc

D = 128
LANES = 16
VPR = D // LANES
NCORES = 2
NSUB = 16
NW = NCORES * NSUB
CHUNK = 128
MAGIC = 8388608.0
INV7 = 1.0 / 7.0
TINY = 1e-30


def _compute_chunk(in_v, out_v):
    @plsc.parallel_loop(0, CHUNK, unroll=2)
    def _(r):
        vs = [in_v[r, pl.ds(j * LANES, LANES)] for j in range(VPR)]
        mn = jnp.minimum(jnp.minimum(jnp.minimum(vs[0], vs[1]),
                                     jnp.minimum(vs[2], vs[3])),
                         jnp.minimum(jnp.minimum(vs[4], vs[5]),
                                     jnp.minimum(vs[6], vs[7])))
        mx = jnp.maximum(jnp.maximum(jnp.maximum(vs[0], vs[1]),
                                     jnp.maximum(vs[2], vs[3])),
                         jnp.maximum(jnp.maximum(vs[4], vs[5]),
                                     jnp.maximum(vs[6], vs[7])))
        mn = jnp.full((LANES,), jnp.min(mn), jnp.float32)
        mx = jnp.full((LANES,), jnp.max(mx), jnp.float32)
        safe = jnp.maximum(mx - mn, TINY)
        inv = 7.0 / safe
        scale = safe * INV7
        for j in range(VPR):
            c = (vs[j] - mn) * inv
            rnd = (c + MAGIC) - MAGIC
            sl = pl.ds(j * LANES, LANES)
            out_v[r, sl] = rnd * scale
            plsc.addupdate(out_v.at[r, sl], mn)


def _sc_body(x_hbm, o_hbm, in0, in1, out0, out1,
             si0, si1, so0, so1):
    rows_per_w = x_hbm.shape[0] // NW
    wid = lax.axis_index("s") * NCORES + lax.axis_index("c")
    base = wid * rows_per_w
    nchunks = rows_per_w // CHUNK
    npairs = nchunks // 2

    def start_in(g, buf, sem):
        pltpu.async_copy(x_hbm.at[pl.ds(base + g * CHUNK, CHUNK)], buf, sem)

    def wait_in(buf, sem):
        pltpu.make_async_copy(x_hbm.at[pl.ds(base, CHUNK)], buf, sem).wait()

    def start_out(g, buf, sem):
        pltpu.async_copy(buf, o_hbm.at[pl.ds(base + g * CHUNK, CHUNK)], sem)

    def wait_out(buf, sem):
        pltpu.make_async_copy(buf, o_hbm.at[pl.ds(base, CHUNK)], sem).wait()

    start_in(0, in0, si0)
    start_in(1, in1, si1)

    def pair(p, carry):
        g = 2 * p

        def phase(g, in_b, out_b, si, so):
            wait_in(in_b, si)

            @pl.when(g >= 2)
            def _():
                wait_out(out_b, so)

            _compute_chunk(in_b, out_b)
            start_out(g, out_b, so)

            @pl.when(g + 2 < nchunks)
            def _():
                start_in(g + 2, in_b, si)

        phase(g, in0, out0, si0, so0)
        phase(g + 1, in1, out1, si1, so1)
        return carry

    lax.fori_loop(0, npairs, pair, 0)
    wait_out(out0, so0)
    wait_out(out1, so1)


def _quantize_recon(x):
    n = x.shape[0]
    mesh = plsc.VectorSubcoreMesh(
        core_axis_name="c", subcore_axis_name="s",
        num_cores=NCORES, num_subcores=NSUB)
    return pl.kernel(
        _sc_body,
        out_type=jax.ShapeDtypeStruct((n, D), jnp.float32),
        mesh=mesh,
        scratch_types=[
            pltpu.VMEM((CHUNK, D), jnp.float32),
            pltpu.VMEM((CHUNK, D), jnp.float32),
            pltpu.VMEM((CHUNK, D), jnp.float32),
            pltpu.VMEM((CHUNK, D), jnp.float32),
            pltpu.SemaphoreType.DMA,
            pltpu.SemaphoreType.DMA,
            pltpu.SemaphoreType.DMA,
            pltpu.SemaphoreType.DMA,
        ],
        compiler_params=pltpu.CompilerParams(needs_layout_passes=False),
    )(x)


def kernel(kv_states):
    batch, num_heads, seq_len, head_dim = kv_states.shape
    x = kv_states.astype(jnp.float32).reshape(-1, head_dim)
    recon = _quantize_recon(x)
    return recon.reshape(batch, num_heads, seq_len, head_dim)
